# Initial kernel scaffold; baseline (speedup 1.0000x reference)
import jax
import jax.numpy as jnp
from jax.experimental import pallas as pl


def kernel(x, weights):
    raise NotImplementedError("write your pallas kernel here")


# speedup vs baseline: 3.7155x; 3.7155x over previous
import jax
import jax.numpy as jnp
from jax import lax
from jax.experimental import pallas as pl

TOP_K = 16
C_BLK = 128
HW = 1024


def _avg_topk_block(x_ref, w_ref, o_ref):
    v = x_ref[0]
    w = w_ref[...]
    col = lax.broadcasted_iota(jnp.int32, (C_BLK, HW), 1)
    acc = jnp.zeros((C_BLK, 1), jnp.float32)
    neg_inf = jnp.float32(-jnp.inf)
    for i in range(TOP_K):
        m = jnp.max(v, axis=1, keepdims=True)
        hit = jnp.where(v == m, col, HW)
        idx = jnp.min(hit, axis=1, keepdims=True)
        v = jnp.where(col == idx, neg_inf, v)
        acc = acc + m * w[:, i : i + 1]
    o_ref[0] = acc


def kernel(x, weights):
    B, C = x.shape[0], x.shape[1]
    x_flat = x.reshape(B, C, HW)
    grid = (B, C // C_BLK)
    out = pl.pallas_call(
        _avg_topk_block,
        grid=grid,
        in_specs=[
            pl.BlockSpec((1, C_BLK, HW), lambda b, c: (b, c, 0)),
            pl.BlockSpec((C_BLK, TOP_K), lambda b, c: (c, 0)),
        ],
        out_specs=pl.BlockSpec((1, C_BLK, 1), lambda b, c: (b, c, 0)),
        out_shape=jax.ShapeDtypeStruct((B, C, 1), jnp.float32),
    )(x_flat, weights)
    return out.reshape(B, C)
